# baseline (speedup 1.0000x reference)
import jax
import jax.numpy as jnp
from jax import lax
from jax.experimental import pallas as pl
from jax.experimental.pallas import tpu as pltpu
from jax.experimental.pallas import tpu_sc as plsc

_NC = 2
_NS = 16
_NW = _NC * _NS


def _topmm_body(q_ref, sk_ref, cv_ref, cf_ref,
                sk_out, cv_out, qn_out, score_out, idx_out, cs_out,
                best_scr, bidx_scr, bcs_scr, sc_scr, cs_scr):
    i = pl.program_id(0)
    nblk = pl.num_programs(0) - 1
    blk = sk_ref.shape[0]

    @pl.when(i == 0)
    def _init():
        q = q_ref[...]
        nrm = jnp.sqrt(jnp.sum(q * q, axis=1, keepdims=True))
        qn_out[...] = q / jnp.maximum(nrm, 1e-12)
        best_scr[...] = jnp.full(best_scr.shape, -jnp.inf, jnp.float32)
        bidx_scr[...] = jnp.zeros(bidx_scr.shape, jnp.int32)
        bcs_scr[...] = jnp.zeros(bcs_scr.shape, jnp.float32)

    @pl.when(i < nblk)
    def _mm():
        qn = qn_out[...]
        sk = sk_ref[...]
        cv = cv_ref[...]
        j = lax.rem(i, 2)
        sc_scr[j] = lax.dot_general(
            qn, sk, (((1,), (1,)), ((), ())),
            preferred_element_type=jnp.float32)
        cs_scr[j] = lax.dot_general(
            cf_ref[...], cv, (((1,), (1,)), ((), ())),
            preferred_element_type=jnp.float32)
        sk_out[...] = sk
        cv_out[...] = cv

    @pl.when(i > 0)
    def _reduce():
        j = lax.rem(i - 1, 2)
        scores = sc_scr[j]
        bm = jnp.max(scores, axis=1)
        col = lax.broadcasted_iota(jnp.int32, scores.shape, 1)
        hit = scores == bm[:, None]
        incol = jnp.min(jnp.where(hit, col, blk), axis=1)
        barg = incol + (i - 1) * blk
        csel = jnp.max(jnp.where(col == incol[:, None], cs_scr[j], -jnp.inf),
                       axis=1)
        better = bm > best_scr[...]
        bidx_scr[...] = jnp.where(better, barg, bidx_scr[...])
        bcs_scr[...] = jnp.where(better, csel, bcs_scr[...])
        best_scr[...] = jnp.where(better, bm, best_scr[...])

    @pl.when(i == nblk)
    def _fin():
        score_out[...] = best_scr[...]
        idx_out[...] = bidx_scr[...]
        cs_out[...] = bcs_scr[...]


def _topmm(query, spatial_key, color_value, color_feat, blk):
    b, feat = query.shape
    mem = spatial_key.shape[0]
    nblk = mem // blk
    last = nblk - 1

    def blkmap(i):
        return (jnp.minimum(i, last), 0)

    return pl.pallas_call(
        _topmm_body,
        grid=(nblk + 1,),
        in_specs=[
            pl.BlockSpec((b, feat), lambda i: (0, 0)),
            pl.BlockSpec((blk, feat), blkmap),
            pl.BlockSpec((blk, feat), blkmap),
            pl.BlockSpec((b, feat), lambda i: (0, 0)),
        ],
        out_specs=[
            pl.BlockSpec((blk, feat), blkmap),
            pl.BlockSpec((blk, feat), blkmap),
            pl.BlockSpec((b, feat), lambda i: (0, 0)),
            pl.BlockSpec((b,), lambda i: (0,)),
            pl.BlockSpec((b,), lambda i: (0,)),
            pl.BlockSpec((b,), lambda i: (0,)),
        ],
        out_shape=[
            jax.ShapeDtypeStruct((mem, feat), jnp.float32),
            jax.ShapeDtypeStruct((mem, feat), jnp.float32),
            jax.ShapeDtypeStruct((b, feat), jnp.float32),
            jax.ShapeDtypeStruct((b,), jnp.float32),
            jax.ShapeDtypeStruct((b,), jnp.int32),
            jax.ShapeDtypeStruct((b,), jnp.float32),
        ],
        scratch_shapes=[
            pltpu.VMEM((b,), jnp.float32),
            pltpu.VMEM((b,), jnp.int32),
            pltpu.VMEM((b,), jnp.float32),
            pltpu.VMEM((2, b, blk), jnp.float32),
            pltpu.VMEM((2, b, blk), jnp.float32),
        ],
    )(query, spatial_key, color_value, color_feat)


def _plan_scatter(w, slots):
    b = w.shape[0]
    qpw = b // _NW
    wmat = (w > 0).reshape(_NW, qpw)
    lane = jnp.arange(qpw, dtype=jnp.int32)
    big = jnp.int32(2 ** 30)
    posm = jnp.where(wmat, lane[None, :], big)
    first = jnp.min(posm, axis=1)
    cnt = jnp.sum(wmat, axis=1).astype(jnp.int32)
    firstc = jnp.where(cnt > 0, first, 0)
    gsrc = jnp.arange(b, dtype=jnp.int32).reshape(_NW, qpw)
    base = (jnp.arange(_NW, dtype=jnp.int32) * qpw + firstc)[:, None]
    src = jnp.where(wmat, gsrc, base)
    slotm = slots.reshape(_NW, qpw)
    slot_first = jnp.take_along_axis(slotm, firstc[:, None], axis=1)
    dst = jnp.where(wmat, slotm, slot_first)
    cntp = jnp.zeros((_NW, 16), jnp.int32).at[:, 0].set(cnt)
    return src, dst, cntp


def _sc_scatter_rows(b, feat):
    qpw = b // _NW
    assert qpw == 32

    def do_one(wid, src_hbm, dst_hbm, cnt_hbm, content_hbm, dst_ref,
               srcbuf, slotbuf, cntbuf, stage, sem):
        pltpu.sync_copy(src_hbm.at[wid], srcbuf)
        pltpu.sync_copy(dst_hbm.at[wid], slotbuf)
        pltpu.sync_copy(cnt_hbm.at[wid], cntbuf)
        cnt = cntbuf[...][0]

        @pl.when(cnt > 0)
        def _():
            pltpu.async_copy(content_hbm.at[srcbuf], stage, sem).wait()
            pltpu.async_copy(stage, dst_ref.at[slotbuf], sem).wait()

    @pl.kernel(
        mesh=plsc.VectorSubcoreMesh(core_axis_name="c", subcore_axis_name="s",
                                    num_cores=_NC, num_subcores=_NS),
        scratch_types=[
            pltpu.VMEM((qpw,), jnp.int32),
            pltpu.VMEM((qpw,), jnp.int32),
            pltpu.VMEM((16,), jnp.int32),
            pltpu.VMEM((qpw, feat), jnp.float32),
            pltpu.SemaphoreType.DMA,
        ],
    )
    def scatter_kernel(sk_ref, cv_ref, upd_hbm, qn_hbm, cf_hbm,
                       src1_hbm, dst1_hbm, cnt1_hbm,
                       src2_hbm, dst2_hbm, cnt2_hbm,
                       srcbuf, slotbuf, cntbuf, stage, sem):
        wid = lax.axis_index("s") * _NC + lax.axis_index("c")
        do_one(wid, src1_hbm, dst1_hbm, cnt1_hbm, upd_hbm, sk_ref,
               srcbuf, slotbuf, cntbuf, stage, sem)
        do_one(wid, src2_hbm, dst2_hbm, cnt2_hbm, qn_hbm, sk_ref,
               srcbuf, slotbuf, cntbuf, stage, sem)
        do_one(wid, src2_hbm, dst2_hbm, cnt2_hbm, cf_hbm, cv_ref,
               srcbuf, slotbuf, cntbuf, stage, sem)

    return scatter_kernel


def kernel(query, color_feat, top_index, color_thres,
           spatial_key, color_value, age, noise):
    b = query.shape[0]
    mem = spatial_key.shape[0]
    feat = query.shape[1]
    blk = min(1024, mem)

    sk_c, cv_c, qn, top1_score, top1_idx, color_sim = _topmm(
        query, spatial_key, color_value, color_feat, blk)

    top1_key = spatial_key[top1_idx]
    memory_mask = color_sim > color_thres
    age1 = age + 1.0

    upd_raw = top1_key + qn
    unrm = jnp.sqrt(jnp.sum(upd_raw * upd_raw, axis=1, keepdims=True))
    upd = upd_raw / jnp.maximum(unrm, 1e-12)

    age1 = age1.at[top1_idx].set(jnp.where(memory_mask, 0.0, age1[top1_idx]))

    unmatched = jnp.logical_not(memory_mask)
    age_with_noise = age1 + noise
    _, old_idx = lax.top_k(age_with_noise, b)

    qiota = lax.iota(jnp.int32, b)
    last_writer = jnp.full((mem,), -1, jnp.int32).at[top1_idx].max(qiota)
    is_last = last_writer[top1_idx] == qiota
    p2marker = jnp.zeros((mem,), jnp.int32).at[old_idx].set(
        unmatched.astype(jnp.int32))
    in_p2 = p2marker[top1_idx] > 0
    w1 = (memory_mask & is_last & ~in_p2).astype(jnp.int32)
    w2 = unmatched.astype(jnp.int32)

    src1, dst1, cnt1 = _plan_scatter(w1, top1_idx)
    src2, dst2, cnt2 = _plan_scatter(w2, old_idx)

    sk_ref = jax.new_ref(sk_c)
    cv_ref = jax.new_ref(cv_c)
    _sc_scatter_rows(b, feat)(sk_ref, cv_ref, upd, qn, color_feat,
                              src1, dst1, cnt1, src2, dst2, cnt2)
    sk3 = jax.freeze(sk_ref)
    cv2 = jax.freeze(cv_ref)

    age2 = age1.at[old_idx].set(jnp.where(unmatched, 0.0, age1[old_idx]))
    mti = jnp.full((mem,), -1, dtype=top_index.dtype)
    mti = mti.at[old_idx].set(jnp.where(unmatched, top_index, mti[old_idx]))
    return sk3, cv2, age2, mti, top1_score


# speedup vs baseline: 1.2779x; 1.2779x over previous
import jax
import jax.numpy as jnp
from jax import lax
from jax.experimental import pallas as pl
from jax.experimental.pallas import tpu as pltpu
from jax.experimental.pallas import tpu_sc as plsc

_NC = 2
_NS = 16
_NW = _NC * _NS


def _topmm_body(q_ref, sk_ref, cv_ref, cf_ref,
                sk_out, cv_out, qn_out, score_out, idx_out, cs_out,
                best_scr, bidx_scr, bcs_scr):
    i = pl.program_id(0)
    nblk = pl.num_programs(0)
    blk = sk_ref.shape[0]

    @pl.when(i == 0)
    def _init():
        q = q_ref[...]
        nrm = jnp.sqrt(jnp.sum(q * q, axis=1, keepdims=True))
        qn_out[...] = q / jnp.maximum(nrm, 1e-12)
        best_scr[...] = jnp.full(best_scr.shape, -jnp.inf, jnp.float32)
        bidx_scr[...] = jnp.zeros(bidx_scr.shape, jnp.int32)
        bcs_scr[...] = jnp.zeros(bcs_scr.shape, jnp.float32)

    qn = qn_out[...]
    sk = sk_ref[...]
    cv = cv_ref[...]
    scores = lax.dot_general(qn, sk, (((1,), (1,)), ((), ())),
                             preferred_element_type=jnp.float32)
    cs = lax.dot_general(cf_ref[...], cv, (((1,), (1,)), ((), ())),
                         preferred_element_type=jnp.float32)
    bm = jnp.max(scores, axis=1)
    col = lax.broadcasted_iota(jnp.int32, scores.shape, 1)
    hit = scores == bm[:, None]
    incol = jnp.min(jnp.where(hit, col, blk), axis=1)
    barg = incol + i * blk
    csel = jnp.max(jnp.where(col == incol[:, None], cs, -jnp.inf), axis=1)
    better = bm > best_scr[...]
    bidx_scr[...] = jnp.where(better, barg, bidx_scr[...])
    bcs_scr[...] = jnp.where(better, csel, bcs_scr[...])
    best_scr[...] = jnp.where(better, bm, best_scr[...])

    sk_out[...] = sk
    cv_out[...] = cv

    @pl.when(i == nblk - 1)
    def _fin():
        score_out[...] = best_scr[...]
        idx_out[...] = bidx_scr[...]
        cs_out[...] = bcs_scr[...]


def _topmm(query, spatial_key, color_value, color_feat, blk):
    b, feat = query.shape
    mem = spatial_key.shape[0]
    nblk = mem // blk

    def blkmap(i):
        return (i, 0)

    return pl.pallas_call(
        _topmm_body,
        grid=(nblk,),
        in_specs=[
            pl.BlockSpec((b, feat), lambda i: (0, 0)),
            pl.BlockSpec((blk, feat), blkmap),
            pl.BlockSpec((blk, feat), blkmap),
            pl.BlockSpec((b, feat), lambda i: (0, 0)),
        ],
        out_specs=[
            pl.BlockSpec((blk, feat), blkmap),
            pl.BlockSpec((blk, feat), blkmap),
            pl.BlockSpec((b, feat), lambda i: (0, 0)),
            pl.BlockSpec((b,), lambda i: (0,)),
            pl.BlockSpec((b,), lambda i: (0,)),
            pl.BlockSpec((b,), lambda i: (0,)),
        ],
        out_shape=[
            jax.ShapeDtypeStruct((mem, feat), jnp.float32),
            jax.ShapeDtypeStruct((mem, feat), jnp.float32),
            jax.ShapeDtypeStruct((b, feat), jnp.float32),
            jax.ShapeDtypeStruct((b,), jnp.float32),
            jax.ShapeDtypeStruct((b,), jnp.int32),
            jax.ShapeDtypeStruct((b,), jnp.float32),
        ],
        scratch_shapes=[
            pltpu.VMEM((b,), jnp.float32),
            pltpu.VMEM((b,), jnp.int32),
            pltpu.VMEM((b,), jnp.float32),
        ],
    )(query, spatial_key, color_value, color_feat)


def _plan_scatter(w, slots):
    b = w.shape[0]
    qpw = b // _NW
    wmat = (w > 0).reshape(_NW, qpw)
    lane = jnp.arange(qpw, dtype=jnp.int32)
    big = jnp.int32(2 ** 30)
    posm = jnp.where(wmat, lane[None, :], big)
    first = jnp.min(posm, axis=1)
    cnt = jnp.sum(wmat, axis=1).astype(jnp.int32)
    firstc = jnp.where(cnt > 0, first, 0)
    gsrc = jnp.arange(b, dtype=jnp.int32).reshape(_NW, qpw)
    base = (jnp.arange(_NW, dtype=jnp.int32) * qpw + firstc)[:, None]
    src = jnp.where(wmat, gsrc, base)
    slotm = slots.reshape(_NW, qpw)
    slot_first = jnp.take_along_axis(slotm, firstc[:, None], axis=1)
    dst = jnp.where(wmat, slotm, slot_first)
    cntp = jnp.zeros((_NW, 16), jnp.int32).at[:, 0].set(cnt)
    return src, dst, cntp


def _sc_scatter_rows(b, feat):
    qpw = b // _NW
    assert qpw == 32

    def do_one(wid, src_hbm, dst_hbm, cnt_hbm, content_hbm, dst_ref,
               srcbuf, slotbuf, cntbuf, stage, sem):
        pltpu.sync_copy(src_hbm.at[wid], srcbuf)
        pltpu.sync_copy(dst_hbm.at[wid], slotbuf)
        pltpu.sync_copy(cnt_hbm.at[wid], cntbuf)
        cnt = cntbuf[...][0]

        @pl.when(cnt > 0)
        def _():
            pltpu.async_copy(content_hbm.at[srcbuf], stage, sem).wait()
            pltpu.async_copy(stage, dst_ref.at[slotbuf], sem).wait()

    @pl.kernel(
        mesh=plsc.VectorSubcoreMesh(core_axis_name="c", subcore_axis_name="s",
                                    num_cores=_NC, num_subcores=_NS),
        scratch_types=[
            pltpu.VMEM((qpw,), jnp.int32),
            pltpu.VMEM((qpw,), jnp.int32),
            pltpu.VMEM((16,), jnp.int32),
            pltpu.VMEM((qpw, feat), jnp.float32),
            pltpu.SemaphoreType.DMA,
        ],
    )
    def scatter_kernel(sk_ref, cv_ref, upd_hbm, qn_hbm, cf_hbm,
                       src1_hbm, dst1_hbm, cnt1_hbm,
                       src2_hbm, dst2_hbm, cnt2_hbm,
                       srcbuf, slotbuf, cntbuf, stage, sem):
        wid = lax.axis_index("s") * _NC + lax.axis_index("c")
        do_one(wid, src1_hbm, dst1_hbm, cnt1_hbm, upd_hbm, sk_ref,
               srcbuf, slotbuf, cntbuf, stage, sem)
        do_one(wid, src2_hbm, dst2_hbm, cnt2_hbm, qn_hbm, sk_ref,
               srcbuf, slotbuf, cntbuf, stage, sem)
        do_one(wid, src2_hbm, dst2_hbm, cnt2_hbm, cf_hbm, cv_ref,
               srcbuf, slotbuf, cntbuf, stage, sem)

    return scatter_kernel


def kernel(query, color_feat, top_index, color_thres,
           spatial_key, color_value, age, noise):
    b = query.shape[0]
    mem = spatial_key.shape[0]
    feat = query.shape[1]
    blk = min(2048, mem)

    sk_c, cv_c, qn, top1_score, top1_idx, color_sim = _topmm(
        query, spatial_key, color_value, color_feat, blk)

    top1_key = spatial_key[top1_idx]
    memory_mask = color_sim > color_thres
    age1 = age + 1.0

    upd_raw = top1_key + qn
    unrm = jnp.sqrt(jnp.sum(upd_raw * upd_raw, axis=1, keepdims=True))
    upd = upd_raw / jnp.maximum(unrm, 1e-12)

    age1 = age1.at[top1_idx].set(jnp.where(memory_mask, 0.0, age1[top1_idx]))

    unmatched = jnp.logical_not(memory_mask)
    age_with_noise = age1 + noise
    _, old_idx = lax.top_k(age_with_noise, b)

    qiota = lax.iota(jnp.int32, b)
    last_writer = jnp.full((mem,), -1, jnp.int32).at[top1_idx].max(qiota)
    is_last = last_writer[top1_idx] == qiota
    p2marker = jnp.zeros((mem,), jnp.int32).at[old_idx].set(
        unmatched.astype(jnp.int32))
    in_p2 = p2marker[top1_idx] > 0
    w1 = (memory_mask & is_last & ~in_p2).astype(jnp.int32)
    w2 = unmatched.astype(jnp.int32)

    src1, dst1, cnt1 = _plan_scatter(w1, top1_idx)
    src2, dst2, cnt2 = _plan_scatter(w2, old_idx)

    sk_ref = jax.new_ref(sk_c)
    cv_ref = jax.new_ref(cv_c)
    _sc_scatter_rows(b, feat)(sk_ref, cv_ref, upd, qn, color_feat,
                              src1, dst1, cnt1, src2, dst2, cnt2)
    sk3 = jax.freeze(sk_ref)
    cv2 = jax.freeze(cv_ref)

    age2 = age1.at[old_idx].set(jnp.where(unmatched, 0.0, age1[old_idx]))
    mti = jnp.full((mem,), -1, dtype=top_index.dtype)
    mti = mti.at[old_idx].set(jnp.where(unmatched, top_index, mti[old_idx]))
    return sk3, cv2, age2, mti, top1_score


# speedup vs baseline: 1.3111x; 1.0259x over previous
import jax
import jax.numpy as jnp
from jax import lax
from jax.experimental import pallas as pl
from jax.experimental.pallas import tpu as pltpu
from jax.experimental.pallas import tpu_sc as plsc

_NC = 2
_NS = 16
_NW = _NC * _NS


def _topmm_body(q_ref, sk_ref, cv_ref,
                sk_out, cv_out, qn_out, score_out, idx_out,
                best_scr, bidx_scr):
    i = pl.program_id(0)
    nblk = pl.num_programs(0)
    blk = sk_ref.shape[0]

    @pl.when(i == 0)
    def _init():
        q = q_ref[...]
        nrm = jnp.sqrt(jnp.sum(q * q, axis=1, keepdims=True))
        qn_out[...] = q / jnp.maximum(nrm, 1e-12)
        best_scr[...] = jnp.full(best_scr.shape, -jnp.inf, jnp.float32)
        bidx_scr[...] = jnp.zeros(bidx_scr.shape, jnp.int32)

    qn = qn_out[...]
    sk = sk_ref[...]
    scores = lax.dot_general(qn, sk, (((1,), (1,)), ((), ())),
                             preferred_element_type=jnp.float32)
    bm = jnp.max(scores, axis=1)
    col = lax.broadcasted_iota(jnp.int32, scores.shape, 1)
    barg = jnp.min(jnp.where(scores == bm[:, None], col, blk), axis=1) + i * blk
    better = bm > best_scr[...]
    bidx_scr[...] = jnp.where(better, barg, bidx_scr[...])
    best_scr[...] = jnp.where(better, bm, best_scr[...])

    sk_out[...] = sk
    cv_out[...] = cv_ref[...]

    @pl.when(i == nblk - 1)
    def _fin():
        score_out[...] = best_scr[...]
        idx_out[...] = bidx_scr[...]


def _topmm(query, spatial_key, color_value, blk):
    b, feat = query.shape
    mem = spatial_key.shape[0]
    nblk = mem // blk

    def blkmap(i):
        return (i, 0)

    return pl.pallas_call(
        _topmm_body,
        grid=(nblk,),
        in_specs=[
            pl.BlockSpec((b, feat), lambda i: (0, 0)),
            pl.BlockSpec((blk, feat), blkmap),
            pl.BlockSpec((blk, feat), blkmap),
        ],
        out_specs=[
            pl.BlockSpec((blk, feat), blkmap),
            pl.BlockSpec((blk, feat), blkmap),
            pl.BlockSpec((b, feat), lambda i: (0, 0)),
            pl.BlockSpec((b,), lambda i: (0,)),
            pl.BlockSpec((b,), lambda i: (0,)),
        ],
        out_shape=[
            jax.ShapeDtypeStruct((mem, feat), jnp.float32),
            jax.ShapeDtypeStruct((mem, feat), jnp.float32),
            jax.ShapeDtypeStruct((b, feat), jnp.float32),
            jax.ShapeDtypeStruct((b,), jnp.float32),
            jax.ShapeDtypeStruct((b,), jnp.int32),
        ],
        scratch_shapes=[
            pltpu.VMEM((b,), jnp.float32),
            pltpu.VMEM((b,), jnp.int32),
        ],
    )(query, spatial_key, color_value)


def _plan_scatter(w, slots):
    b = w.shape[0]
    qpw = b // _NW
    wmat = (w > 0).reshape(_NW, qpw)
    lane = jnp.arange(qpw, dtype=jnp.int32)
    big = jnp.int32(2 ** 30)
    posm = jnp.where(wmat, lane[None, :], big)
    first = jnp.min(posm, axis=1)
    cnt = jnp.sum(wmat, axis=1).astype(jnp.int32)
    firstc = jnp.where(cnt > 0, first, 0)
    gsrc = jnp.arange(b, dtype=jnp.int32).reshape(_NW, qpw)
    base = (jnp.arange(_NW, dtype=jnp.int32) * qpw + firstc)[:, None]
    src = jnp.where(wmat, gsrc, base)
    slotm = slots.reshape(_NW, qpw)
    slot_first = jnp.take_along_axis(slotm, firstc[:, None], axis=1)
    dst = jnp.where(wmat, slotm, slot_first)
    cntp = jnp.zeros((_NW, 16), jnp.int32).at[:, 0].set(cnt)
    return src, dst, cntp


def _sc_scatter_rows(b, feat):
    qpw = b // _NW
    assert qpw == 32

    def do_one(wid, src_hbm, dst_hbm, cnt_hbm, content_hbm, dst_ref,
               srcbuf, slotbuf, cntbuf, stage, sem):
        pltpu.sync_copy(src_hbm.at[wid], srcbuf)
        pltpu.sync_copy(dst_hbm.at[wid], slotbuf)
        pltpu.sync_copy(cnt_hbm.at[wid], cntbuf)
        cnt = cntbuf[...][0]

        @pl.when(cnt > 0)
        def _():
            pltpu.async_copy(content_hbm.at[srcbuf], stage, sem).wait()
            pltpu.async_copy(stage, dst_ref.at[slotbuf], sem).wait()

    @pl.kernel(
        mesh=plsc.VectorSubcoreMesh(core_axis_name="c", subcore_axis_name="s",
                                    num_cores=_NC, num_subcores=_NS),
        scratch_types=[
            pltpu.VMEM((qpw,), jnp.int32),
            pltpu.VMEM((qpw,), jnp.int32),
            pltpu.VMEM((16,), jnp.int32),
            pltpu.VMEM((qpw, feat), jnp.float32),
            pltpu.SemaphoreType.DMA,
        ],
    )
    def scatter_kernel(sk_ref, cv_ref, upd_hbm, qn_hbm, cf_hbm,
                       src1_hbm, dst1_hbm, cnt1_hbm,
                       src2_hbm, dst2_hbm, cnt2_hbm,
                       srcbuf, slotbuf, cntbuf, stage, sem):
        wid = lax.axis_index("s") * _NC + lax.axis_index("c")
        do_one(wid, src1_hbm, dst1_hbm, cnt1_hbm, upd_hbm, sk_ref,
               srcbuf, slotbuf, cntbuf, stage, sem)
        do_one(wid, src2_hbm, dst2_hbm, cnt2_hbm, qn_hbm, sk_ref,
               srcbuf, slotbuf, cntbuf, stage, sem)
        do_one(wid, src2_hbm, dst2_hbm, cnt2_hbm, cf_hbm, cv_ref,
               srcbuf, slotbuf, cntbuf, stage, sem)

    return scatter_kernel


def kernel(query, color_feat, top_index, color_thres,
           spatial_key, color_value, age, noise):
    b = query.shape[0]
    mem = spatial_key.shape[0]
    feat = query.shape[1]
    blk = min(2048, mem)

    sk_c, cv_c, qn, top1_score, top1_idx = _topmm(
        query, spatial_key, color_value, blk)

    top1_key = spatial_key[top1_idx]
    top1_cv = color_value[top1_idx]
    color_sim = jnp.sum(top1_cv * color_feat, axis=1)
    memory_mask = color_sim > color_thres
    age1 = age + 1.0

    upd_raw = top1_key + qn
    unrm = jnp.sqrt(jnp.sum(upd_raw * upd_raw, axis=1, keepdims=True))
    upd = upd_raw / jnp.maximum(unrm, 1e-12)

    age1 = age1.at[top1_idx].set(jnp.where(memory_mask, 0.0, age1[top1_idx]))

    unmatched = jnp.logical_not(memory_mask)
    age_with_noise = age1 + noise
    _, old_idx = lax.top_k(age_with_noise, b)

    qiota = lax.iota(jnp.int32, b)
    last_writer = jnp.full((mem,), -1, jnp.int32).at[top1_idx].max(qiota)
    is_last = last_writer[top1_idx] == qiota
    p2marker = jnp.zeros((mem,), jnp.int32).at[old_idx].set(
        unmatched.astype(jnp.int32))
    in_p2 = p2marker[top1_idx] > 0
    w1 = (memory_mask & is_last & ~in_p2).astype(jnp.int32)
    w2 = unmatched.astype(jnp.int32)

    src1, dst1, cnt1 = _plan_scatter(w1, top1_idx)
    src2, dst2, cnt2 = _plan_scatter(w2, old_idx)

    sk_ref = jax.new_ref(sk_c)
    cv_ref = jax.new_ref(cv_c)
    _sc_scatter_rows(b, feat)(sk_ref, cv_ref, upd, qn, color_feat,
                              src1, dst1, cnt1, src2, dst2, cnt2)
    sk3 = jax.freeze(sk_ref)
    cv2 = jax.freeze(cv_ref)

    age2 = age1.at[old_idx].set(jnp.where(unmatched, 0.0, age1[old_idx]))
    mti = jnp.full((mem,), -1, dtype=top_index.dtype)
    mti = mti.at[old_idx].set(jnp.where(unmatched, top_index, mti[old_idx]))
    return sk3, cv2, age2, mti, top1_score


# speedup vs baseline: 1.5319x; 1.1684x over previous
import jax
import jax.numpy as jnp
from jax import lax
from jax.experimental import pallas as pl
from jax.experimental.pallas import tpu as pltpu
from jax.experimental.pallas import tpu_sc as plsc

_NC = 2
_NS = 16
_NW = _NC * _NS


def _topmm_body(q_ref, sk_ref, cv_ref,
                sk_out, cv_out, qn_out, score_out, idx_out,
                best_scr, bidx_scr, sc_scr):
    i = pl.program_id(0)
    nblk = pl.num_programs(0) - 1
    blk = sk_ref.shape[0]

    @pl.when(i == 0)
    def _init():
        q = q_ref[...]
        nrm = jnp.sqrt(jnp.sum(q * q, axis=1, keepdims=True))
        qn_out[...] = q / jnp.maximum(nrm, 1e-12)
        best_scr[...] = jnp.full(best_scr.shape, -jnp.inf, jnp.float32)
        bidx_scr[...] = jnp.zeros(bidx_scr.shape, jnp.int32)

    @pl.when(i < nblk)
    def _mm():
        qn = qn_out[...]
        sk = sk_ref[...]
        sc_scr[lax.rem(i, 2)] = lax.dot_general(
            qn, sk, (((1,), (1,)), ((), ())),
            preferred_element_type=jnp.float32)
        sk_out[...] = sk
        cv_out[...] = cv_ref[...]

    @pl.when(i > 0)
    def _reduce():
        scores = sc_scr[lax.rem(i - 1, 2)]
        bm = jnp.max(scores, axis=1)
        col = lax.broadcasted_iota(jnp.int32, scores.shape, 1)
        barg = jnp.min(jnp.where(scores == bm[:, None], col, blk),
                       axis=1) + (i - 1) * blk
        better = bm > best_scr[...]
        bidx_scr[...] = jnp.where(better, barg, bidx_scr[...])
        best_scr[...] = jnp.where(better, bm, best_scr[...])

    @pl.when(i == nblk)
    def _fin():
        score_out[...] = best_scr[...]
        idx_out[...] = bidx_scr[...]


def _topmm(query, spatial_key, color_value, blk):
    b, feat = query.shape
    mem = spatial_key.shape[0]
    nblk = mem // blk
    last = nblk - 1

    def blkmap(i):
        return (jnp.minimum(i, last), 0)

    return pl.pallas_call(
        _topmm_body,
        grid=(nblk + 1,),
        in_specs=[
            pl.BlockSpec((b, feat), lambda i: (0, 0)),
            pl.BlockSpec((blk, feat), blkmap),
            pl.BlockSpec((blk, feat), blkmap),
        ],
        out_specs=[
            pl.BlockSpec((blk, feat), blkmap),
            pl.BlockSpec((blk, feat), blkmap),
            pl.BlockSpec((b, feat), lambda i: (0, 0)),
            pl.BlockSpec((b,), lambda i: (0,)),
            pl.BlockSpec((b,), lambda i: (0,)),
        ],
        out_shape=[
            jax.ShapeDtypeStruct((mem, feat), jnp.float32),
            jax.ShapeDtypeStruct((mem, feat), jnp.float32),
            jax.ShapeDtypeStruct((b, feat), jnp.float32),
            jax.ShapeDtypeStruct((b,), jnp.float32),
            jax.ShapeDtypeStruct((b,), jnp.int32),
        ],
        scratch_shapes=[
            pltpu.VMEM((b,), jnp.float32),
            pltpu.VMEM((b,), jnp.int32),
            pltpu.VMEM((2, b, blk), jnp.float32),
        ],
    )(query, spatial_key, color_value)


def _plan_scatter(w, slots):
    b = w.shape[0]
    qpw = b // _NW
    wmat = (w > 0).reshape(_NW, qpw)
    lane = jnp.arange(qpw, dtype=jnp.int32)
    big = jnp.int32(2 ** 30)
    posm = jnp.where(wmat, lane[None, :], big)
    first = jnp.min(posm, axis=1)
    cnt = jnp.sum(wmat, axis=1).astype(jnp.int32)
    firstc = jnp.where(cnt > 0, first, 0)
    gsrc = jnp.arange(b, dtype=jnp.int32).reshape(_NW, qpw)
    base = (jnp.arange(_NW, dtype=jnp.int32) * qpw + firstc)[:, None]
    src = jnp.where(wmat, gsrc, base)
    slotm = slots.reshape(_NW, qpw)
    slot_first = jnp.take_along_axis(slotm, firstc[:, None], axis=1)
    dst = jnp.where(wmat, slotm, slot_first)
    cntp = jnp.zeros((_NW, 16), jnp.int32).at[:, 0].set(cnt)
    return src, dst, cntp


def _sc_scatter_rows(b, feat):
    qpw = b // _NW
    assert qpw == 32

    def do_one(wid, src_hbm, dst_hbm, cnt_hbm, content_hbm, dst_ref,
               srcbuf, slotbuf, cntbuf, stage, sem):
        pltpu.sync_copy(src_hbm.at[wid], srcbuf)
        pltpu.sync_copy(dst_hbm.at[wid], slotbuf)
        pltpu.sync_copy(cnt_hbm.at[wid], cntbuf)
        cnt = cntbuf[...][0]

        @pl.when(cnt > 0)
        def _():
            pltpu.async_copy(content_hbm.at[srcbuf], stage, sem).wait()
            pltpu.async_copy(stage, dst_ref.at[slotbuf], sem).wait()

    @pl.kernel(
        mesh=plsc.VectorSubcoreMesh(core_axis_name="c", subcore_axis_name="s",
                                    num_cores=_NC, num_subcores=_NS),
        scratch_types=[
            pltpu.VMEM((qpw,), jnp.int32),
            pltpu.VMEM((qpw,), jnp.int32),
            pltpu.VMEM((16,), jnp.int32),
            pltpu.VMEM((qpw, feat), jnp.float32),
            pltpu.SemaphoreType.DMA,
        ],
    )
    def scatter_kernel(sk_ref, cv_ref, upd_hbm, qn_hbm, cf_hbm,
                       src1_hbm, dst1_hbm, cnt1_hbm,
                       src2_hbm, dst2_hbm, cnt2_hbm,
                       srcbuf, slotbuf, cntbuf, stage, sem):
        wid = lax.axis_index("s") * _NC + lax.axis_index("c")
        do_one(wid, src1_hbm, dst1_hbm, cnt1_hbm, upd_hbm, sk_ref,
               srcbuf, slotbuf, cntbuf, stage, sem)
        do_one(wid, src2_hbm, dst2_hbm, cnt2_hbm, qn_hbm, sk_ref,
               srcbuf, slotbuf, cntbuf, stage, sem)
        do_one(wid, src2_hbm, dst2_hbm, cnt2_hbm, cf_hbm, cv_ref,
               srcbuf, slotbuf, cntbuf, stage, sem)

    return scatter_kernel


def kernel(query, color_feat, top_index, color_thres,
           spatial_key, color_value, age, noise):
    b = query.shape[0]
    mem = spatial_key.shape[0]
    feat = query.shape[1]
    blk = min(2048, mem)

    sk_c, cv_c, qn, top1_score, top1_idx = _topmm(
        query, spatial_key, color_value, blk)

    top1_key = spatial_key[top1_idx]
    top1_cv = color_value[top1_idx]
    color_sim = jnp.sum(top1_cv * color_feat, axis=1)
    memory_mask = color_sim > color_thres
    age1 = age + 1.0

    upd_raw = top1_key + qn
    unrm = jnp.sqrt(jnp.sum(upd_raw * upd_raw, axis=1, keepdims=True))
    upd = upd_raw / jnp.maximum(unrm, 1e-12)

    age1 = age1.at[top1_idx].set(jnp.where(memory_mask, 0.0, age1[top1_idx]))

    unmatched = jnp.logical_not(memory_mask)
    age_with_noise = age1 + noise
    _, old_idx = lax.top_k(age_with_noise, b)

    qiota = lax.iota(jnp.int32, b)
    eq = top1_idx[None, :] == top1_idx[:, None]
    has_later = jnp.any(eq & (qiota[None, :] > qiota[:, None]), axis=1)
    in_p2 = jnp.any((old_idx[None, :] == top1_idx[:, None]) & unmatched[None, :],
                    axis=1)
    w1 = (memory_mask & ~has_later & ~in_p2).astype(jnp.int32)
    w2 = unmatched.astype(jnp.int32)

    src1, dst1, cnt1 = _plan_scatter(w1, top1_idx)
    src2, dst2, cnt2 = _plan_scatter(w2, old_idx)

    sk_ref = jax.new_ref(sk_c)
    cv_ref = jax.new_ref(cv_c)
    _sc_scatter_rows(b, feat)(sk_ref, cv_ref, upd, qn, color_feat,
                              src1, dst1, cnt1, src2, dst2, cnt2)
    sk3 = jax.freeze(sk_ref)
    cv2 = jax.freeze(cv_ref)

    age2 = age1.at[old_idx].set(jnp.where(unmatched, 0.0, age1[old_idx]))
    mti = jnp.full((mem,), -1, dtype=top_index.dtype)
    mti = mti.at[old_idx].set(jnp.where(unmatched, top_index, mti[old_idx]))
    return sk3, cv2, age2, mti, top1_score
